# Initial kernel scaffold; baseline (speedup 1.0000x reference)
import jax
import jax.numpy as jnp
from jax.experimental import pallas as pl


def kernel(indices, table):
    raise NotImplementedError("write your pallas kernel here")


# speedup vs baseline: 6.9276x; 6.9276x over previous
import jax
import jax.numpy as jnp
from jax import lax
from jax.experimental import pallas as pl
from jax.experimental.pallas import tpu as pltpu
from jax.experimental.pallas import tpu_sc as plsc

V = 1000
D = 128
B = 16384
S = 200
N = B * S
NC = 2
NS = 16
NW = NC * NS
PER_W = N // NW
CHUNK = 128
GROUP = 2 * CHUNK
BLOCK = 2 * GROUP
IDX_ROWS = BLOCK // 128
N_BLOCKS = PER_W // BLOCK


def _emb_body(idx_hbm, table_hbm, out_hbm, ibuf, rows,
              isem0, isem1, gsem0, gsem1, ssem0, ssem1):
    wid = lax.axis_index("s") * NC + lax.axis_index("c")
    wrow = wid * (PER_W // 128)
    wout = wid * PER_W

    isems = (isem0, isem1)
    gsems = (gsem0, gsem1)
    ssems = (ssem0, ssem1)

    pltpu.sync_copy(idx_hbm.at[pl.ds(wrow, IDX_ROWS)], ibuf.at[0])

    def body(k2, carry):
        k = k2 * 2
        for b in range(2):
            blk = k + b
            if b == 0:
                @pl.when(k > 0)
                def _wait_idx():
                    pltpu.make_async_copy(
                        idx_hbm.at[pl.ds(wrow, IDX_ROWS)],
                        ibuf.at[b], isems[b]).wait()
            else:
                pltpu.make_async_copy(
                    idx_hbm.at[pl.ds(wrow, IDX_ROWS)],
                    ibuf.at[b], isems[b]).wait()
            nxt = blk + 1
            if b == 0:
                pltpu.make_async_copy(
                    idx_hbm.at[pl.ds(wrow + nxt * IDX_ROWS, IDX_ROWS)],
                    ibuf.at[1 - b], isems[1 - b]).start()
            else:
                @pl.when(k < N_BLOCKS - 2)
                def _pref_idx():
                    pltpu.make_async_copy(
                        idx_hbm.at[pl.ds(wrow + nxt * IDX_ROWS, IDX_ROWS)],
                        ibuf.at[1 - b], isems[1 - b]).start()
            for g in range(2):
                gi = blk * 2 + g
                if b == 0:
                    @pl.when(k > 0)
                    def _wait_store():
                        pltpu.make_async_copy(
                            rows.at[g], out_hbm.at[pl.ds(wout, GROUP)],
                            ssems[g]).wait()
                else:
                    pltpu.make_async_copy(
                        rows.at[g], out_hbm.at[pl.ds(wout, GROUP)],
                        ssems[g]).wait()
                c0 = pltpu.make_async_copy(
                    table_hbm.at[ibuf.at[b, 2 * g]],
                    rows.at[g, pl.ds(0, CHUNK)], gsems[g])
                c1 = pltpu.make_async_copy(
                    table_hbm.at[ibuf.at[b, 2 * g + 1]],
                    rows.at[g, pl.ds(CHUNK, CHUNK)], gsems[g])
                c0.start()
                c1.start()
                c0.wait()
                c1.wait()
                pltpu.make_async_copy(
                    rows.at[g],
                    out_hbm.at[pl.ds(wout + gi * GROUP, GROUP)],
                    ssems[g]).start()
        return carry

    lax.fori_loop(0, N_BLOCKS // 2, body, None)

    for g in range(2):
        last = (N_BLOCKS * 2 - 2) + g
        pltpu.make_async_copy(
            rows.at[g],
            out_hbm.at[pl.ds(wout + last * GROUP, GROUP)],
            ssems[g]).wait()


def kernel(indices, table):
    idx2d = indices.reshape(N // 128, 128)
    mesh = plsc.VectorSubcoreMesh(core_axis_name="c", subcore_axis_name="s")
    f = pl.kernel(
        _emb_body,
        mesh=mesh,
        out_type=jax.ShapeDtypeStruct((N, D), jnp.float32),
        scratch_types=[
            pltpu.VMEM((2, IDX_ROWS, 128), jnp.int32),
            pltpu.VMEM((2, GROUP, D), jnp.float32),
            pltpu.SemaphoreType.DMA, pltpu.SemaphoreType.DMA,
            pltpu.SemaphoreType.DMA, pltpu.SemaphoreType.DMA,
            pltpu.SemaphoreType.DMA, pltpu.SemaphoreType.DMA,
        ],
    )
    out = f(idx2d, table)
    return out.reshape(B, S, D)


# speedup vs baseline: 19.0008x; 2.7428x over previous
import jax
import jax.numpy as jnp
from jax import lax
from jax.experimental import pallas as pl
from jax.experimental.pallas import tpu as pltpu
from jax.experimental.pallas import tpu_sc as plsc

V = 1000
D = 128
B = 16384
S = 200
N = B * S
NC = 2
NS = 16
NW = NC * NS
PER_W = N // NW
CHUNK = 128
GROUP = 2 * CHUNK
BLOCK = 2 * GROUP
IDX_ROWS = BLOCK // 128
N_BLOCKS = PER_W // BLOCK


def _emb_body(idx_hbm, table_hbm, out_hbm, table_sp, ibuf, rows,
              isem0, isem1, gsem0, gsem1, ssem0, ssem1):
    sub = lax.axis_index("s")
    wid = sub * NC + lax.axis_index("c")
    wrow = wid * (PER_W // 128)
    wout = wid * PER_W

    isems = (isem0, isem1)
    gsems = (gsem0, gsem1)
    ssems = (ssem0, ssem1)

    @pl.when(sub == 0)
    def _load_table():
        pltpu.sync_copy(table_hbm, table_sp)
    plsc.subcore_barrier()

    pltpu.sync_copy(idx_hbm.at[pl.ds(wrow, IDX_ROWS)], ibuf.at[0])

    def body(k2, carry):
        k = k2 * 2
        for b in range(2):
            blk = k + b
            if b == 0:
                @pl.when(k > 0)
                def _wait_idx():
                    pltpu.make_async_copy(
                        idx_hbm.at[pl.ds(wrow, IDX_ROWS)],
                        ibuf.at[b], isems[b]).wait()
            else:
                pltpu.make_async_copy(
                    idx_hbm.at[pl.ds(wrow, IDX_ROWS)],
                    ibuf.at[b], isems[b]).wait()
            nxt = blk + 1
            if b == 0:
                pltpu.make_async_copy(
                    idx_hbm.at[pl.ds(wrow + nxt * IDX_ROWS, IDX_ROWS)],
                    ibuf.at[1 - b], isems[1 - b]).start()
            else:
                @pl.when(k < N_BLOCKS - 2)
                def _pref_idx():
                    pltpu.make_async_copy(
                        idx_hbm.at[pl.ds(wrow + nxt * IDX_ROWS, IDX_ROWS)],
                        ibuf.at[1 - b], isems[1 - b]).start()
            for g in range(2):
                gi = blk * 2 + g
                if b == 0:
                    @pl.when(k > 0)
                    def _wait_store():
                        pltpu.make_async_copy(
                            rows.at[g], out_hbm.at[pl.ds(wout, GROUP)],
                            ssems[g]).wait()
                else:
                    pltpu.make_async_copy(
                        rows.at[g], out_hbm.at[pl.ds(wout, GROUP)],
                        ssems[g]).wait()
                c0 = pltpu.make_async_copy(
                    table_sp.at[ibuf.at[b, 2 * g]],
                    rows.at[g, pl.ds(0, CHUNK)], gsems[g])
                c1 = pltpu.make_async_copy(
                    table_sp.at[ibuf.at[b, 2 * g + 1]],
                    rows.at[g, pl.ds(CHUNK, CHUNK)], gsems[g])
                c0.start()
                c1.start()
                c0.wait()
                c1.wait()
                pltpu.make_async_copy(
                    rows.at[g],
                    out_hbm.at[pl.ds(wout + gi * GROUP, GROUP)],
                    ssems[g]).start()
        return carry

    lax.fori_loop(0, N_BLOCKS // 2, body, None)

    for g in range(2):
        last = (N_BLOCKS * 2 - 2) + g
        pltpu.make_async_copy(
            rows.at[g],
            out_hbm.at[pl.ds(wout + last * GROUP, GROUP)],
            ssems[g]).wait()


def kernel(indices, table):
    idx2d = indices.reshape(N // 128, 128)
    mesh = plsc.VectorSubcoreMesh(core_axis_name="c", subcore_axis_name="s")
    f = pl.kernel(
        _emb_body,
        mesh=mesh,
        out_type=jax.ShapeDtypeStruct((N, D), jnp.float32),
        scratch_types=[
            pltpu.VMEM_SHARED((V, D), jnp.float32),
            pltpu.VMEM((2, IDX_ROWS, 128), jnp.int32),
            pltpu.VMEM((2, GROUP, D), jnp.float32),
            pltpu.SemaphoreType.DMA, pltpu.SemaphoreType.DMA,
            pltpu.SemaphoreType.DMA, pltpu.SemaphoreType.DMA,
            pltpu.SemaphoreType.DMA, pltpu.SemaphoreType.DMA,
        ],
    )
    out = f(idx2d, table)
    return out.reshape(B, S, D)


# speedup vs baseline: 19.5777x; 1.0304x over previous
import jax
import jax.numpy as jnp
from jax import lax
from jax.experimental import pallas as pl
from jax.experimental.pallas import tpu as pltpu
from jax.experimental.pallas import tpu_sc as plsc

V = 1000
D = 128
B = 16384
S = 200
N = B * S
NC = 2
NS = 16
NW = NC * NS
PER_W = N // NW
CHUNK = 128
NBUF = 4
BLOCK = 4 * CHUNK
IDX_ROWS = BLOCK // 128
N_BLOCKS = PER_W // BLOCK
N_GROUPS = PER_W // CHUNK


def _emb_body(idx_hbm, table_hbm, out_hbm, table_sp, ibuf, rows,
              isem0, isem1,
              gsem0, gsem1, gsem2, gsem3,
              ssem0, ssem1, ssem2, ssem3):
    sub = lax.axis_index("s")
    wid = sub * NC + lax.axis_index("c")
    wrow = wid * (PER_W // 128)
    wout = wid * PER_W

    isems = (isem0, isem1)
    gsems = (gsem0, gsem1, gsem2, gsem3)
    ssems = (ssem0, ssem1, ssem2, ssem3)

    @pl.when(sub == 0)
    def _load_table():
        pltpu.sync_copy(table_hbm, table_sp)
    plsc.subcore_barrier()

    pltpu.sync_copy(idx_hbm.at[pl.ds(wrow, IDX_ROWS)], ibuf.at[0])

    def _gather(bi, g, buf):
        return pltpu.make_async_copy(
            table_sp.at[ibuf.at[bi, g]], rows.at[buf], gsems[buf])

    def _store(gi, buf):
        return pltpu.make_async_copy(
            rows.at[buf], out_hbm.at[pl.ds(wout + gi * CHUNK, CHUNK)],
            ssems[buf])

    def body(k2, carry):
        k = k2 * 2
        for b in range(2):
            blk = k + b
            if b == 0:
                @pl.when(k > 0)
                def _wait_idx():
                    pltpu.make_async_copy(
                        idx_hbm.at[pl.ds(wrow, IDX_ROWS)],
                        ibuf.at[b], isems[b]).wait()
            else:
                pltpu.make_async_copy(
                    idx_hbm.at[pl.ds(wrow, IDX_ROWS)],
                    ibuf.at[b], isems[b]).wait()
            nxt = blk + 1
            for g in range(4):
                gi = blk * 4 + g
                if g == 2:
                    if b == 0:
                        pltpu.make_async_copy(
                            idx_hbm.at[pl.ds(wrow + nxt * IDX_ROWS, IDX_ROWS)],
                            ibuf.at[1 - b], isems[1 - b]).start()
                    else:
                        @pl.when(k < N_BLOCKS - 2)
                        def _pref_idx():
                            pltpu.make_async_copy(
                                idx_hbm.at[pl.ds(wrow + nxt * IDX_ROWS,
                                                 IDX_ROWS)],
                                ibuf.at[1 - b], isems[1 - b]).start()
                if b == 0:
                    @pl.when(k > 0)
                    def _wait_store():
                        _store(gi - NBUF, g).wait()
                else:
                    _store(gi - NBUF, g).wait()
                _gather(b, g, g).start()
                pg = (g + 2) % 4
                if b == 0 and g < 2:
                    @pl.when(k > 0)
                    def _store_prev():
                        _gather(0, 0, pg).wait()
                        _store(gi - 2, pg).start()
                else:
                    _gather(0, 0, pg).wait()
                    _store(gi - 2, pg).start()
        return carry

    lax.fori_loop(0, N_BLOCKS // 2, body, None)

    for gi in (N_GROUPS - 2, N_GROUPS - 1):
        _gather(0, 0, gi % 4).wait()
        _store(gi, gi % 4).start()
    for gi in range(N_GROUPS - 4, N_GROUPS):
        _store(gi, gi % 4).wait()


def kernel(indices, table):
    idx2d = indices.reshape(N // 128, 128)
    mesh = plsc.VectorSubcoreMesh(core_axis_name="c", subcore_axis_name="s")
    f = pl.kernel(
        _emb_body,
        mesh=mesh,
        out_type=jax.ShapeDtypeStruct((N, D), jnp.float32),
        scratch_types=[
            pltpu.VMEM_SHARED((V, D), jnp.float32),
            pltpu.VMEM((2, IDX_ROWS, 128), jnp.int32),
            pltpu.VMEM((NBUF, CHUNK, D), jnp.float32),
            pltpu.SemaphoreType.DMA, pltpu.SemaphoreType.DMA,
            pltpu.SemaphoreType.DMA, pltpu.SemaphoreType.DMA,
            pltpu.SemaphoreType.DMA, pltpu.SemaphoreType.DMA,
            pltpu.SemaphoreType.DMA, pltpu.SemaphoreType.DMA,
            pltpu.SemaphoreType.DMA, pltpu.SemaphoreType.DMA,
        ],
    )
    out = f(idx2d, table)
    return out.reshape(B, S, D)
